# baseline (speedup 1.0000x reference)
import jax
import jax.numpy as jnp
import numpy as np
from jax import lax

_HI = jax.lax.Precision.HIGHEST
from jax.experimental import pallas as pl
from jax.experimental.pallas import tpu as pltpu
from jax.experimental.pallas import tpu_sc as plsc

N_NODES = 100000
N_EDGES = 6400000
K = 8
NC, NS, L = 2, 16, 16
NPAD = 100352
RPT = NPAD // NS
CHUNK = 128
SB = 8
NSB = N_EDGES // (CHUNK * SB)

DEG_BASE = NSB // NS
DEG_EXTRA = NSB - DEG_BASE * NS

SC_SB = NSB // NC
MAIN_BASE = SC_SB // NS
MAIN_EXTRA = SC_SB - MAIN_BASE * NS


def _mesh():
    return plsc.VectorSubcoreMesh(
        core_axis_name="c", subcore_axis_name="s",
        num_cores=NC, num_subcores=NS)


_SC_PARAMS = pltpu.CompilerParams(needs_layout_passes=False,
                                  use_tc_tiling_on_sc=False)


def _deg_body(e_row, e_col, deg, idxbuf, hist, dsem):
    cid = lax.axis_index("c")
    sid = lax.axis_index("s")
    ones = jnp.full((L,), 1.0, jnp.float32)
    zeros = jnp.zeros((L,), jnp.float32)

    def zbody(i, c):
        hist[pl.ds(i * L, L)] = zeros
        return c
    lax.fori_loop(0, NPAD // L, zbody, 0)

    def accumulate(src):
        base = sid * DEG_BASE + jnp.minimum(sid, DEG_EXTRA)
        cnt = DEG_BASE + jnp.where(sid < DEG_EXTRA, 1, 0)
        pltpu.async_copy(src.at[pl.ds(base * SB, SB)], idxbuf.at[0], dsem)

        def trip(t, c):
            p = t & 1
            pltpu.make_async_copy(src.at[pl.ds(0, SB)], idxbuf.at[p],
                                  dsem).wait()

            @pl.when(t + 1 < cnt)
            def _():
                pltpu.async_copy(src.at[pl.ds((base + t + 1) * SB, SB)],
                                 idxbuf.at[1 - p], dsem)
            for j in range(SB):
                for v in range(CHUNK // L):
                    idx = idxbuf[p, j, pl.ds(v * L, L)]
                    plsc.addupdate_scatter(hist, [idx], ones)
            return c
        lax.fori_loop(0, cnt, trip, 0)

    @pl.when(cid == 0)
    def _():
        accumulate(e_col)

    @pl.when(cid == 1)
    def _():
        accumulate(e_row)

    pltpu.sync_copy(hist, deg.at[pl.ds((cid * NS + sid) * NPAD, NPAD)])


def _deg_call(e_row, e_col):
    return pl.kernel(
        _deg_body,
        out_type=jax.ShapeDtypeStruct((NC * NS * NPAD,), jnp.float32),
        mesh=_mesh(),
        compiler_params=_SC_PARAMS,
        scratch_types=[
            pltpu.VMEM((2, SB, CHUNK), jnp.int32),
            pltpu.VMEM((NPAD,), jnp.float32),
            pltpu.SemaphoreType.DMA,
        ],
    )(e_row, e_col)


RPK = NPAD * K // 128
R128 = NPAD // 128

_REP = np.zeros((128, 1024), np.float32)
for _b in range(128):
    _REP[_b, 8 * _b:8 * _b + 8] = 1.0

_AIL = np.zeros((2, 128, 128), np.float32)
_BIL = np.zeros((2, 128, 128), np.float32)
for _e in range(2):
    for _m in range(8):
        for _j in range(8):
            _AIL[_e, 64 * _e + 8 * _m + _j, 16 * _m + _j] = 1.0
            _BIL[_e, 64 * _e + 8 * _m + _j, 16 * _m + 8 + _j] = 1.0

_BPX = RPK // 8


def _mm_body(x_ref, bdin_ref, bdout_ref, xwin_ref, xwout_ref):
    xr = jnp.maximum(x_ref[...], 0.0)
    xwin_ref[...] = jnp.dot(xr, bdin_ref[...], precision=_HI,
                            preferred_element_type=jnp.float32)
    xwout_ref[...] = jnp.dot(xr, bdout_ref[...], precision=_HI,
                             preferred_element_type=jnp.float32)


def _mm_call(xpk, bd_in, bd_out):
    return pl.pallas_call(
        _mm_body,
        grid=(8,),
        in_specs=[
            pl.BlockSpec((_BPX, 128), lambda i: (i, 0)),
            pl.BlockSpec((128, 128), lambda i: (0, 0)),
            pl.BlockSpec((128, 128), lambda i: (0, 0)),
        ],
        out_specs=(
            pl.BlockSpec((_BPX, 128), lambda i: (i, 0)),
            pl.BlockSpec((_BPX, 128), lambda i: (i, 0)),
        ),
        out_shape=(
            jax.ShapeDtypeStruct((RPK, 128), jnp.float32),
            jax.ShapeDtypeStruct((RPK, 128), jnp.float32),
        ),
    )(xpk, bd_in, bd_out)


_BRD = 112
_GPD = R128 // _BRD
_BPY = _BRD * 8


def _point_body(deg_ref, xwin_ref, xwout_ref, rep_ref,
                yin_ref, yout_ref, dpk_ref):
    deg = jnp.sum(deg_ref[...], axis=1)
    dinv = lax.rsqrt(deg + 1.0)
    rep = rep_ref[...]
    dpk = []
    for d in range(NC):
        e = jnp.dot(dinv[d], rep, precision=_HI,
                    preferred_element_type=jnp.float32)
        dpk.append(e.reshape(_BRD, 8, 128).reshape(_BPY, 128))
    dpk_ref[...] = jnp.stack(dpk, axis=0)
    yin_ref[...] = xwin_ref[...] * dpk[0]
    yout_ref[...] = xwout_ref[...] * dpk[1]


def _point_call(deg4, xwin, xwout):
    rep = jnp.asarray(_REP)
    return pl.pallas_call(
        _point_body,
        grid=(_GPD,),
        in_specs=[
            pl.BlockSpec((NC, NS, _BRD, 128), lambda i: (0, 0, i, 0)),
            pl.BlockSpec((_BPY, 128), lambda i: (i, 0)),
            pl.BlockSpec((_BPY, 128), lambda i: (i, 0)),
            pl.BlockSpec((128, 1024), lambda i: (0, 0)),
        ],
        out_specs=(
            pl.BlockSpec((_BPY, 128), lambda i: (i, 0)),
            pl.BlockSpec((_BPY, 128), lambda i: (i, 0)),
            pl.BlockSpec((NC, _BPY, 128), lambda i: (0, i, 0)),
        ),
        out_shape=(
            jax.ShapeDtypeStruct((RPK, 128), jnp.float32),
            jax.ShapeDtypeStruct((RPK, 128), jnp.float32),
            jax.ShapeDtypeStruct((NC, RPK, 128), jnp.float32),
        ),
    )(deg4, xwin, xwout, rep)


def _main_body(e_row, e_col, yin, yout, zrows, accs,
               idx_g, idx_s, gbuf, stage, acc, gsem, ssem, lsem):
    cid = lax.axis_index("c")
    sid = lax.axis_index("s")
    myslice = pl.ds(sid * RPT, RPT)

    pltpu.sync_copy(zrows.at[myslice], stage)
    pltpu.sync_copy(stage, acc.at[myslice])
    plsc.subcore_barrier()

    base = sid * DEG_BASE + jnp.minimum(sid, DEG_EXTRA)
    cnt = DEG_BASE + jnp.where(sid < DEG_EXTRA, 1, 0)

    def refill(s, slot, eg, es):
        pltpu.async_copy(eg.at[pl.ds((base + s) * SB, SB)], idx_g.at[slot],
                         lsem)
        pltpu.async_copy(es.at[pl.ds((base + s) * SB, SB)], idx_s.at[slot],
                         lsem)

    def run(table, eg, es):
        refill(0, 0, eg, es)

        def trip(s, c):
            p = s & 1
            pltpu.make_async_copy(eg.at[pl.ds(0, SB)], idx_g.at[p],
                                  lsem).wait()
            pltpu.make_async_copy(eg.at[pl.ds(0, SB)], idx_s.at[p],
                                  lsem).wait()

            @pl.when(s + 1 < cnt)
            def _():
                refill(s + 1, 1 - p, eg, es)

            @pl.when(s > 0)
            def _():
                for h in range(SB):
                    pltpu.make_async_copy(zrows.at[pl.ds(0, CHUNK)],
                                          gbuf.at[p, h], ssem).wait()

            gh = [pltpu.async_copy(table.at[idx_g.at[p, h]], gbuf.at[p, h],
                                   gsem) for h in range(SB)]
            for h in range(SB):
                gh[h].wait()
                pltpu.async_copy(gbuf.at[p, h], acc.at[idx_s.at[p, h]],
                                 ssem, add=True)
            return c
        lax.fori_loop(0, cnt, trip, 0)

        def fdrain(h, c):
            pltpu.make_async_copy(zrows.at[pl.ds(0, CHUNK)],
                                  gbuf.at[0, 0], ssem).wait()
            return c
        lax.fori_loop(0, SB, fdrain, 0)

    @pl.when(cid == 0)
    def _():
        run(yin, e_row, e_col)

    @pl.when(cid == 1)
    def _():
        run(yout, e_col, e_row)

    plsc.subcore_barrier()
    pltpu.sync_copy(acc.at[myslice], stage)
    pltpu.sync_copy(stage, accs.at[cid, myslice])


def _main_call(e_row, e_col, yin, yout, zrows):
    return pl.kernel(
        _main_body,
        out_type=jax.ShapeDtypeStruct((NC, NPAD, K), jnp.float32),
        mesh=_mesh(),
        compiler_params=_SC_PARAMS,
        scratch_types=[
            pltpu.VMEM((2, SB, CHUNK), jnp.int32),
            pltpu.VMEM((2, SB, CHUNK), jnp.int32),
            pltpu.VMEM((2, SB, CHUNK, K), jnp.float32),
            pltpu.VMEM((RPT, K), jnp.float32),
            pltpu.VMEM_SHARED((NPAD, K), jnp.float32),
            pltpu.SemaphoreType.DMA,
            pltpu.SemaphoreType.DMA,
            pltpu.SemaphoreType.DMA,
        ],
    )(e_row, e_col, yin, yout, zrows)


def _final_body(part_ref, yin_ref, yout_ref, dpk_ref, bin_ref, bout_ref,
                ail_ref, bil_ref, out_ref):
    in_x = dpk_ref[0] * (part_ref[0] + yin_ref[...]) + bin_ref[...]
    out_x = dpk_ref[1] * (part_ref[1] + yout_ref[...]) + bout_ref[...]
    rows = []
    for e in range(2):
        rows.append(
            jnp.dot(in_x, ail_ref[e], precision=_HI,
                    preferred_element_type=jnp.float32)
            + jnp.dot(out_x, bil_ref[e], precision=_HI,
                      preferred_element_type=jnp.float32))
    out_ref[...] = jnp.stack(rows, axis=1).reshape(2 * _BPX, 128)


def _final_call(part, yin, yout, dpk, b_in, b_out):
    bin_t = jnp.tile(b_in, 16).reshape(1, 128)
    bout_t = jnp.tile(b_out, 16).reshape(1, 128)
    return pl.pallas_call(
        _final_body,
        grid=(8,),
        in_specs=[
            pl.BlockSpec((NC, _BPX, 128), lambda i: (0, i, 0)),
            pl.BlockSpec((_BPX, 128), lambda i: (i, 0)),
            pl.BlockSpec((_BPX, 128), lambda i: (i, 0)),
            pl.BlockSpec((NC, _BPX, 128), lambda i: (0, i, 0)),
            pl.BlockSpec((1, 128), lambda i: (0, 0)),
            pl.BlockSpec((1, 128), lambda i: (0, 0)),
            pl.BlockSpec((2, 128, 128), lambda i: (0, 0, 0)),
            pl.BlockSpec((2, 128, 128), lambda i: (0, 0, 0)),
        ],
        out_specs=pl.BlockSpec((2 * _BPX, 128), lambda i: (i, 0)),
        out_shape=jax.ShapeDtypeStruct((2 * RPK, 128), jnp.float32),
    )(part, yin, yout, dpk, bin_t, bout_t,
      jnp.asarray(_AIL), jnp.asarray(_BIL))


def kernel(x, edge_index, W_in, b_in, W_out, b_out):
    e = edge_index.astype(jnp.int32)
    e_row = e[0].reshape(N_EDGES // CHUNK, CHUNK)
    e_col = e[1].reshape(N_EDGES // CHUNK, CHUNK)
    xpk = jnp.pad(x.reshape(N_NODES * K // 128, 128),
                  ((0, RPK - N_NODES * K // 128), (0, 0)))
    eye = jnp.eye(16, dtype=jnp.float32)
    xwin, xwout = _mm_call(xpk, jnp.kron(eye, W_in), jnp.kron(eye, W_out))
    deg4 = _deg_call(e_row, e_col).reshape(NC, NS, R128, 128)
    yin_pk, yout_pk, dpk = _point_call(deg4, xwin, xwout)
    zrows = jnp.zeros((NPAD, K), jnp.float32)
    part = _main_call(e_row, e_col, yin_pk.reshape(NPAD, K),
                      yout_pk.reshape(NPAD, K), zrows)
    out_pk = _final_call(part.reshape(NC, RPK, 128), yin_pk, yout_pk, dpk,
                         b_in, b_out)
    return out_pk.reshape(NPAD, 2 * K)[:N_NODES]


# speedup vs baseline: 1.0196x; 1.0196x over previous
import jax
import jax.numpy as jnp
import numpy as np
from jax import lax

_HI = jax.lax.Precision.HIGHEST
from jax.experimental import pallas as pl
from jax.experimental.pallas import tpu as pltpu
from jax.experimental.pallas import tpu_sc as plsc

N_NODES = 100000
N_EDGES = 6400000
K = 8
NC, NS, L = 2, 16, 16
NPAD = 100352
RPT = NPAD // NS
CHUNK = 128
SB = 8
NSB = N_EDGES // (CHUNK * SB)

DEG_BASE = NSB // NS
DEG_EXTRA = NSB - DEG_BASE * NS

SC_SB = NSB // NC
MAIN_BASE = SC_SB // NS
MAIN_EXTRA = SC_SB - MAIN_BASE * NS


def _mesh():
    return plsc.VectorSubcoreMesh(
        core_axis_name="c", subcore_axis_name="s",
        num_cores=NC, num_subcores=NS)


_SC_PARAMS = pltpu.CompilerParams(needs_layout_passes=False,
                                  use_tc_tiling_on_sc=False)


def _deg_body(e_row, e_col, deg, idxbuf, hist, dsem):
    cid = lax.axis_index("c")
    sid = lax.axis_index("s")
    ones = jnp.full((L,), 1.0, jnp.float32)
    zeros = jnp.zeros((L,), jnp.float32)

    def zbody(i, c):
        hist[pl.ds(i * L, L)] = zeros
        return c
    lax.fori_loop(0, NPAD // L, zbody, 0)

    def accumulate(src):
        base = sid * DEG_BASE + jnp.minimum(sid, DEG_EXTRA)
        cnt = DEG_BASE + jnp.where(sid < DEG_EXTRA, 1, 0)
        pltpu.async_copy(src.at[pl.ds(base * SB, SB)], idxbuf.at[0], dsem)

        def trip(t, c):
            p = t & 1
            pltpu.make_async_copy(src.at[pl.ds(0, SB)], idxbuf.at[p],
                                  dsem).wait()

            @pl.when(t + 1 < cnt)
            def _():
                pltpu.async_copy(src.at[pl.ds((base + t + 1) * SB, SB)],
                                 idxbuf.at[1 - p], dsem)
            for j in range(SB):
                for v in range(CHUNK // L):
                    idx = idxbuf[p, j, pl.ds(v * L, L)]
                    plsc.addupdate_scatter(hist, [idx], ones)
            return c
        lax.fori_loop(0, cnt, trip, 0)

    @pl.when(cid == 0)
    def _():
        accumulate(e_col)

    @pl.when(cid == 1)
    def _():
        accumulate(e_row)

    pltpu.sync_copy(hist, deg.at[pl.ds((cid * NS + sid) * NPAD, NPAD)])


def _deg_call(e_row, e_col):
    return pl.kernel(
        _deg_body,
        out_type=jax.ShapeDtypeStruct((NC * NS * NPAD,), jnp.float32),
        mesh=_mesh(),
        compiler_params=_SC_PARAMS,
        scratch_types=[
            pltpu.VMEM((2, SB, CHUNK), jnp.int32),
            pltpu.VMEM((NPAD,), jnp.float32),
            pltpu.SemaphoreType.DMA,
        ],
    )(e_row, e_col)


RPK = NPAD * K // 128
R128 = NPAD // 128

_REP = np.zeros((128, 1024), np.float32)
for _b in range(128):
    _REP[_b, 8 * _b:8 * _b + 8] = 1.0

_AIL = np.zeros((2, 128, 128), np.float32)
_BIL = np.zeros((2, 128, 128), np.float32)
for _e in range(2):
    for _m in range(8):
        for _j in range(8):
            _AIL[_e, 64 * _e + 8 * _m + _j, 16 * _m + _j] = 1.0
            _BIL[_e, 64 * _e + 8 * _m + _j, 16 * _m + 8 + _j] = 1.0

_BPX = RPK // 8


def _mm_body(x_ref, bdin_ref, bdout_ref, xwin_ref, xwout_ref):
    xr = jnp.maximum(x_ref[...], 0.0)
    xwin_ref[...] = jnp.dot(xr, bdin_ref[...], precision=_HI,
                            preferred_element_type=jnp.float32)
    xwout_ref[...] = jnp.dot(xr, bdout_ref[...], precision=_HI,
                             preferred_element_type=jnp.float32)


def _mm_call(xpk, bd_in, bd_out):
    return pl.pallas_call(
        _mm_body,
        grid=(8,),
        in_specs=[
            pl.BlockSpec((_BPX, 128), lambda i: (i, 0)),
            pl.BlockSpec((128, 128), lambda i: (0, 0)),
            pl.BlockSpec((128, 128), lambda i: (0, 0)),
        ],
        out_specs=(
            pl.BlockSpec((_BPX, 128), lambda i: (i, 0)),
            pl.BlockSpec((_BPX, 128), lambda i: (i, 0)),
        ),
        out_shape=(
            jax.ShapeDtypeStruct((RPK, 128), jnp.float32),
            jax.ShapeDtypeStruct((RPK, 128), jnp.float32),
        ),
    )(xpk, bd_in, bd_out)


_BRD = 112
_GPD = R128 // _BRD
_BPY = _BRD * 8


def _point_body(deg_ref, xwin_ref, xwout_ref, rep_ref,
                yin_ref, yout_ref, dpk_ref):
    deg = jnp.sum(deg_ref[...], axis=1)
    dinv = lax.rsqrt(deg + 1.0)
    rep = rep_ref[...]
    dpk = []
    for d in range(NC):
        e = jnp.dot(dinv[d], rep, precision=_HI,
                    preferred_element_type=jnp.float32)
        dpk.append(e.reshape(_BRD, 8, 128).reshape(_BPY, 128))
    dpk_ref[...] = jnp.stack(dpk, axis=0)
    yin_ref[...] = xwin_ref[...] * dpk[0]
    yout_ref[...] = xwout_ref[...] * dpk[1]


def _point_call(deg4, xwin, xwout):
    rep = jnp.asarray(_REP)
    return pl.pallas_call(
        _point_body,
        grid=(_GPD,),
        in_specs=[
            pl.BlockSpec((NC, NS, _BRD, 128), lambda i: (0, 0, i, 0)),
            pl.BlockSpec((_BPY, 128), lambda i: (i, 0)),
            pl.BlockSpec((_BPY, 128), lambda i: (i, 0)),
            pl.BlockSpec((128, 1024), lambda i: (0, 0)),
        ],
        out_specs=(
            pl.BlockSpec((_BPY, 128), lambda i: (i, 0)),
            pl.BlockSpec((_BPY, 128), lambda i: (i, 0)),
            pl.BlockSpec((NC, _BPY, 128), lambda i: (0, i, 0)),
        ),
        out_shape=(
            jax.ShapeDtypeStruct((RPK, 128), jnp.float32),
            jax.ShapeDtypeStruct((RPK, 128), jnp.float32),
            jax.ShapeDtypeStruct((NC, RPK, 128), jnp.float32),
        ),
    )(deg4, xwin, xwout, rep)


def _main_body(e_row, e_col, e_row1, e_col1, yin, yout, zrows, accs,
               idx_g, idx_s, gbuf, stage, acc, gsem, ssem, lsem):
    cid = lax.axis_index("c")
    sid = lax.axis_index("s")
    myslice = pl.ds(sid * RPT, RPT)

    pltpu.sync_copy(zrows.at[myslice], stage)
    pltpu.sync_copy(stage, acc.at[myslice])
    plsc.subcore_barrier()

    base = sid * DEG_BASE + jnp.minimum(sid, DEG_EXTRA)
    cnt = DEG_BASE + jnp.where(sid < DEG_EXTRA, 1, 0)

    def refill(s, slot, eg, es):
        pltpu.async_copy(eg.at[pl.ds(base + s, 1)], idx_g.at[slot], lsem)
        pltpu.async_copy(es.at[pl.ds((base + s) * SB, SB)], idx_s.at[slot],
                         lsem)

    def run(table, eg, es):
        refill(0, 0, eg, es)

        def trip(s, c):
            p = s & 1
            pltpu.make_async_copy(eg.at[pl.ds(0, 1)], idx_g.at[p],
                                  lsem).wait()
            pltpu.make_async_copy(es.at[pl.ds(0, SB)], idx_s.at[p],
                                  lsem).wait()
            gh = pltpu.async_copy(table.at[idx_g.at[p, 0]], gbuf.at[p],
                                  gsem)

            @pl.when(s + 1 < cnt)
            def _():
                refill(s + 1, 1 - p, eg, es)

            @pl.when(s > 0)
            def _():
                for h in range(SB):
                    pltpu.make_async_copy(zrows.at[pl.ds(0, CHUNK)],
                                          gbuf.at[p, pl.ds(0, CHUNK)],
                                          ssem).wait()
            gh.wait()
            for h in range(SB):
                pltpu.async_copy(gbuf.at[p, pl.ds(h * CHUNK, CHUNK)],
                                 acc.at[idx_s.at[p, h]], ssem, add=True)
            return c
        lax.fori_loop(0, cnt, trip, 0)

        def fdrain(h, c):
            pltpu.make_async_copy(zrows.at[pl.ds(0, CHUNK)],
                                  gbuf.at[0, pl.ds(0, CHUNK)], ssem).wait()
            return c
        lax.fori_loop(0, SB, fdrain, 0)

    @pl.when(cid == 0)
    def _():
        run(yin, e_row1, e_col)

    @pl.when(cid == 1)
    def _():
        run(yout, e_col1, e_row)

    plsc.subcore_barrier()
    pltpu.sync_copy(acc.at[myslice], stage)
    pltpu.sync_copy(stage, accs.at[cid, myslice])


def _main_call(e_row, e_col, e_row1, e_col1, yin, yout, zrows):
    return pl.kernel(
        _main_body,
        out_type=jax.ShapeDtypeStruct((NC, NPAD, K), jnp.float32),
        mesh=_mesh(),
        compiler_params=_SC_PARAMS,
        scratch_types=[
            pltpu.VMEM((2, 1, SB * CHUNK), jnp.int32),
            pltpu.VMEM((2, SB, CHUNK), jnp.int32),
            pltpu.VMEM((2, SB * CHUNK, K), jnp.float32),
            pltpu.VMEM((RPT, K), jnp.float32),
            pltpu.VMEM_SHARED((NPAD, K), jnp.float32),
            pltpu.SemaphoreType.DMA,
            pltpu.SemaphoreType.DMA,
            pltpu.SemaphoreType.DMA,
        ],
    )(e_row, e_col, e_row1, e_col1, yin, yout, zrows)


def _final_body(part_ref, yin_ref, yout_ref, dpk_ref, bin_ref, bout_ref,
                ail_ref, bil_ref, out_ref):
    in_x = dpk_ref[0] * (part_ref[0] + yin_ref[...]) + bin_ref[...]
    out_x = dpk_ref[1] * (part_ref[1] + yout_ref[...]) + bout_ref[...]
    rows = []
    for e in range(2):
        rows.append(
            jnp.dot(in_x, ail_ref[e], precision=_HI,
                    preferred_element_type=jnp.float32)
            + jnp.dot(out_x, bil_ref[e], precision=_HI,
                      preferred_element_type=jnp.float32))
    out_ref[...] = jnp.stack(rows, axis=1).reshape(2 * _BPX, 128)


def _final_call(part, yin, yout, dpk, b_in, b_out):
    bin_t = jnp.tile(b_in, 16).reshape(1, 128)
    bout_t = jnp.tile(b_out, 16).reshape(1, 128)
    return pl.pallas_call(
        _final_body,
        grid=(8,),
        in_specs=[
            pl.BlockSpec((NC, _BPX, 128), lambda i: (0, i, 0)),
            pl.BlockSpec((_BPX, 128), lambda i: (i, 0)),
            pl.BlockSpec((_BPX, 128), lambda i: (i, 0)),
            pl.BlockSpec((NC, _BPX, 128), lambda i: (0, i, 0)),
            pl.BlockSpec((1, 128), lambda i: (0, 0)),
            pl.BlockSpec((1, 128), lambda i: (0, 0)),
            pl.BlockSpec((2, 128, 128), lambda i: (0, 0, 0)),
            pl.BlockSpec((2, 128, 128), lambda i: (0, 0, 0)),
        ],
        out_specs=pl.BlockSpec((2 * _BPX, 128), lambda i: (i, 0)),
        out_shape=jax.ShapeDtypeStruct((2 * RPK, 128), jnp.float32),
    )(part, yin, yout, dpk, bin_t, bout_t,
      jnp.asarray(_AIL), jnp.asarray(_BIL))


def kernel(x, edge_index, W_in, b_in, W_out, b_out):
    e = edge_index.astype(jnp.int32)
    e_row = e[0].reshape(N_EDGES // CHUNK, CHUNK)
    e_col = e[1].reshape(N_EDGES // CHUNK, CHUNK)
    xpk = jnp.pad(x.reshape(N_NODES * K // 128, 128),
                  ((0, RPK - N_NODES * K // 128), (0, 0)))
    eye = jnp.eye(16, dtype=jnp.float32)
    xwin, xwout = _mm_call(xpk, jnp.kron(eye, W_in), jnp.kron(eye, W_out))
    deg4 = _deg_call(e_row, e_col).reshape(NC, NS, R128, 128)
    yin_pk, yout_pk, dpk = _point_call(deg4, xwin, xwout)
    zrows = jnp.zeros((NPAD, K), jnp.float32)
    e2 = jax.lax.optimization_barrier(e)
    e_row1 = e2[0].reshape(NSB, SB * CHUNK)
    e_col1 = e2[1].reshape(NSB, SB * CHUNK)
    part = _main_call(e_row, e_col, e_row1, e_col1,
                      yin_pk.reshape(NPAD, K), yout_pk.reshape(NPAD, K),
                      zrows)
    out_pk = _final_call(part.reshape(NC, RPK, 128), yin_pk, yout_pk, dpk,
                         b_in, b_out)
    return out_pk.reshape(NPAD, 2 * K)[:N_NODES]
